# baseline (speedup 1.0000x reference)
import functools

import jax
import jax.numpy as jnp
from jax import lax
from jax.experimental import pallas as pl
from jax.experimental.pallas import tpu as pltpu, tpu_sc as plsc

BATCH = 16384
EMB_DIM = 64
NUM_WORKERS = 32
B_PER_W = BATCH // NUM_WORKERS
CHUNK = 128
N_CHUNKS = B_PER_W // CHUNK


def _transe_gather(h, r, t, ent_emb, rel_emb):
    mesh = plsc.VectorSubcoreMesh(core_axis_name="c", subcore_axis_name="s")

    out_sd = jax.ShapeDtypeStruct((BATCH, EMB_DIM), jnp.float32)
    row_buf = lambda: pltpu.VMEM((CHUNK, EMB_DIM), jnp.float32)

    @functools.partial(
        pl.kernel,
        mesh=mesh,
        compiler_params=pltpu.CompilerParams(use_tc_tiling_on_sc=False,
                                             needs_layout_passes=False,
                                             disable_bounds_checks=True),
        out_type=[out_sd, out_sd, out_sd],
        scratch_types=[
            [[row_buf(), row_buf(), row_buf()] for _ in range(2)],
            [pltpu.VMEM((B_PER_W,), jnp.int32) for _ in range(3)],
            pltpu.SemaphoreType.DMA,
            pltpu.SemaphoreType.DMA,
        ],
    )
    def k(h_hbm, r_hbm, t_hbm, ent_hbm, rel_hbm,
          h_out, r_out, t_out,
          rows, vidx, gsem, wsem):
        wid = lax.axis_index("s") * 2 + lax.axis_index("c")
        base = wid * B_PER_W
        tabs = (ent_hbm, rel_hbm, ent_hbm)
        outs = (h_out, r_out, t_out)

        for j, src in enumerate((h_hbm, r_hbm, t_hbm)):
            pltpu.sync_copy(src.at[pl.ds(base, B_PER_W)], vidx[j])

        for c in range(N_CHUNKS):
            b = c % 2
            off = base + c * CHUNK
            sl = pl.ds(off, CHUNK)

            if c >= 2:
                for j in range(3):
                    pltpu.make_async_copy(
                        rows[b][j], outs[j].at[sl], wsem).wait()

            def fire(g, _, b=b):
                for j in range(3):
                    ev = vidx[j][pl.ds(c * CHUNK + g * 16, 16)]
                    for lane in range(16):
                        pltpu.async_copy(tabs[j].at[ev[lane]],
                                         rows[b][j].at[g * 16 + lane], gsem)
                return 0
            lax.fori_loop(0, CHUNK // 16, fire, 0)

            def drain(i, _, b=b):
                for j in range(3):
                    pltpu.make_async_copy(tabs[j].at[0], rows[b][j].at[0],
                                          gsem).wait()
                return 0
            lax.fori_loop(0, CHUNK, drain, 0)
            for j in range(3):
                pltpu.async_copy(rows[b][j], outs[j].at[sl], wsem)

        for c in range(max(0, N_CHUNKS - 2), N_CHUNKS):
            b = c % 2
            sl = pl.ds(base + c * CHUNK, CHUNK)
            for j in range(3):
                pltpu.make_async_copy(rows[b][j], outs[j].at[sl], wsem).wait()

    return k(h, r, t, ent_emb, rel_emb)


def kernel(h, r, t, ent_emb, rel_emb):
    h = h.astype(jnp.int32)
    r = r.astype(jnp.int32)
    t = t.astype(jnp.int32)
    return tuple(_transe_gather(h, r, t, ent_emb, rel_emb))


# speedup vs baseline: 1.0008x; 1.0008x over previous
import functools

import jax
import jax.numpy as jnp
from jax import lax
from jax.experimental import pallas as pl
from jax.experimental.pallas import tpu as pltpu, tpu_sc as plsc

BATCH = 16384
EMB_DIM = 64
NUM_WORKERS = 32
B_PER_W = BATCH // NUM_WORKERS
CHUNK = 128
N_CHUNKS = B_PER_W // CHUNK
TC_BLOCK = 2048


def _transe_gather(h, r, t, ent_emb, rel_emb):
    mesh = plsc.VectorSubcoreMesh(core_axis_name="c", subcore_axis_name="s")

    out_sd = jax.ShapeDtypeStruct((BATCH, EMB_DIM), jnp.float32)
    row_buf = lambda: pltpu.VMEM((CHUNK, EMB_DIM), jnp.float32)

    @functools.partial(
        pl.kernel,
        mesh=mesh,
        compiler_params=pltpu.CompilerParams(use_tc_tiling_on_sc=False,
                                             needs_layout_passes=False,
                                             disable_bounds_checks=True),
        out_type=[out_sd, out_sd, out_sd],
        scratch_types=[
            [[row_buf(), row_buf(), row_buf()] for _ in range(2)],
            [pltpu.VMEM((B_PER_W,), jnp.int32) for _ in range(3)],
            pltpu.SemaphoreType.DMA,
            pltpu.SemaphoreType.DMA,
        ],
    )
    def k(h_hbm, r_hbm, t_hbm, ent_hbm, rel_hbm,
          h_out, r_out, t_out,
          rows, vidx, gsem, wsem):
        wid = lax.axis_index("s") * 2 + lax.axis_index("c")
        base = wid * B_PER_W
        tabs = (ent_hbm, rel_hbm, ent_hbm)
        outs = (h_out, r_out, t_out)

        for j, src in enumerate((h_hbm, r_hbm, t_hbm)):
            pltpu.sync_copy(src.at[pl.ds(base, B_PER_W)], vidx[j])

        for c in range(N_CHUNKS):
            b = c % 2
            sl = pl.ds(base + c * CHUNK, CHUNK)

            if c >= 2:
                for j in range(3):
                    pltpu.make_async_copy(
                        rows[b][j], outs[j].at[sl], wsem).wait()

            def fire(g, _, b=b):
                for j in range(3):
                    ev = vidx[j][pl.ds(c * CHUNK + g * 16, 16)]
                    for lane in range(16):
                        pltpu.async_copy(tabs[j].at[ev[lane]],
                                         rows[b][j].at[g * 16 + lane], gsem)
                return 0
            lax.fori_loop(0, CHUNK // 16, fire, 0)

            def drain(i, _, b=b):
                for j in range(3):
                    pltpu.make_async_copy(tabs[j].at[0], rows[b][j].at[0],
                                          gsem).wait()
                return 0
            lax.fori_loop(0, CHUNK, drain, 0)
            for j in range(3):
                pltpu.async_copy(rows[b][j], outs[j].at[sl], wsem)

        for c in range(max(0, N_CHUNKS - 2), N_CHUNKS):
            b = c % 2
            sl = pl.ds(base + c * CHUNK, CHUNK)
            for j in range(3):
                pltpu.make_async_copy(rows[b][j], outs[j].at[sl], wsem).wait()

    return k(h, r, t, ent_emb, rel_emb)


def _tc_transpose3(a, b, c):
    def body(a_ref, b_ref, c_ref, at_ref, bt_ref, ct_ref):
        at_ref[...] = a_ref[...].T
        bt_ref[...] = b_ref[...].T
        ct_ref[...] = c_ref[...].T

    out_t = jax.ShapeDtypeStruct((EMB_DIM, BATCH), jnp.float32)
    in_spec = pl.BlockSpec((TC_BLOCK, EMB_DIM), lambda i: (i, 0))
    out_spec = pl.BlockSpec((EMB_DIM, TC_BLOCK), lambda i: (0, i))
    return pl.pallas_call(
        body,
        grid=(BATCH // TC_BLOCK,),
        in_specs=[in_spec] * 3,
        out_specs=[out_spec] * 3,
        out_shape=[out_t, out_t, out_t],
    )(a, b, c)


def kernel(h, r, t, ent_emb, rel_emb):
    h = h.astype(jnp.int32)
    r = r.astype(jnp.int32)
    t = t.astype(jnp.int32)
    h_g, r_g, t_g = _transe_gather(h, r, t, ent_emb, rel_emb)
    h_t, r_t, t_t = _tc_transpose3(h_g, r_g, t_g)
    return (h_t.T, r_t.T, t_t.T)
